# baseline (speedup 1.0000x reference)
import jax
import jax.numpy as jnp
import numpy as np
from jax.experimental import pallas as pl
from jax.experimental.pallas import tpu as pltpu


def _fused_kernel(b_dim, nelem,
                  xyz1t_ref, xyz2t_ref, x2_ref, w_ref, b_ref, g_ref, bt_ref,
                  out_ref, yt_scr, sum_scr, sq_scr):
    i = pl.program_id(0)

    @pl.when(i < b_dim)
    def _compute():
        xyz1 = xyz1t_ref[0]
        xyz2t = xyz2t_ref[0]
        n1 = xyz1.shape[0]
        n2 = xyz2t.shape[1]
        ab = jax.lax.dot_general(
            xyz1.astype(jnp.bfloat16), xyz2t.astype(jnp.bfloat16),
            (((1,), (0,)), ((), ())), preferred_element_type=jnp.float32)
        b2 = (xyz2t[0] * xyz2t[0] + xyz2t[1] * xyz2t[1]
              + xyz2t[2] * xyz2t[2])[None, :]
        a2 = (xyz1[:, 0] * xyz1[:, 0] + xyz1[:, 1] * xyz1[:, 1]
              + xyz1[:, 2] * xyz1[:, 2])[:, None]
        d_sel = (-2.0 * ab + a2) + b2

        cols = jax.lax.broadcasted_iota(
            jnp.int32, (n1, n2), 1).astype(jnp.float32)
        big = jnp.float32(n2)
        dcur = d_sel
        recips = []
        masks = []
        for _ in range(3):
            dk = jnp.min(dcur, axis=1, keepdims=True)
            ik = jnp.min(jnp.where(dcur == dk, cols, big), axis=1,
                         keepdims=True)
            mk = cols == ik
            recips.append(1.0 / (dk + 1e-8))
            masks.append(mk)
            dcur = jnp.where(mk, jnp.float32(np.inf), dcur)
        norm = recips[0] + recips[1] + recips[2]
        s_mat = jnp.zeros((n1, n2), jnp.float32)
        for k in range(3):
            s_mat = s_mat + jnp.where(masks[k], recips[k] / norm, 0.0)

        a_mat = jax.lax.dot_general(
            w_ref[...].astype(jnp.bfloat16), s_mat.astype(jnp.bfloat16),
            (((1,), (0,)), ((), ())),
            preferred_element_type=jnp.float32)
        yt = jax.lax.dot_general(
            a_mat.astype(jnp.bfloat16), x2_ref[0].astype(jnp.bfloat16),
            (((1,), (0,)), ((), ())),
            preferred_element_type=jnp.float32)
        yt = yt + b_ref[...]
        yt_scr[i] = yt
        sum_scr[pl.ds(i, 1), :] = jnp.sum(yt, axis=0, keepdims=True)
        sq_scr[pl.ds(i, 1), :] = jnp.sum(yt * yt, axis=0, keepdims=True)

    @pl.when(i >= b_dim)
    def _bn():
        bidx = i - b_dim
        s = jnp.sum(sum_scr[...], axis=0, keepdims=True)
        ss = jnp.sum(sq_scr[...], axis=0, keepdims=True)
        mean = s / nelem
        var = ss / nelem - mean * mean
        scale = g_ref[...] / jnp.sqrt(var + 1e-5)
        shift = bt_ref[...] - mean * scale
        out_ref[0] = jnp.maximum(yt_scr[bidx] * scale + shift, 0.0)


def kernel(x1, x2, xyz1, xyz2, fc1_w, fc1_b, bn_gamma, bn_beta):
    del x1
    b_dim, n2, c2 = x2.shape
    n1 = xyz1.shape[1]
    out_ch = fc1_w.shape[0]

    xyz2t = jnp.swapaxes(xyz2, 1, 2)
    bias_col = fc1_b[:, None]
    nelem = float(b_dim * out_ch)

    def body(*refs):
        return _fused_kernel(b_dim, nelem, *refs)

    last = b_dim - 1
    out = pl.pallas_call(
        body,
        grid=(2 * b_dim,),
        in_specs=[
            pl.BlockSpec((1, n1, 3), lambda i: (jnp.minimum(i, last), 0, 0)),
            pl.BlockSpec((1, 3, n2), lambda i: (jnp.minimum(i, last), 0, 0)),
            pl.BlockSpec((1, n2, c2), lambda i: (jnp.minimum(i, last), 0, 0)),
            pl.BlockSpec((out_ch, n1), lambda i: (0, 0)),
            pl.BlockSpec((out_ch, 1), lambda i: (0, 0)),
            pl.BlockSpec((1, c2), lambda i: (0, 0)),
            pl.BlockSpec((1, c2), lambda i: (0, 0)),
        ],
        out_specs=pl.BlockSpec(
            (1, out_ch, c2),
            lambda i: (jnp.maximum(i - (last + 1), 0), 0, 0)),
        out_shape=jax.ShapeDtypeStruct((b_dim, out_ch, c2), jnp.float32),
        scratch_shapes=[
            pltpu.VMEM((b_dim, out_ch, c2), jnp.float32),
            pltpu.VMEM((b_dim, c2), jnp.float32),
            pltpu.VMEM((b_dim, c2), jnp.float32),
        ],
        compiler_params=pltpu.CompilerParams(
            dimension_semantics=("arbitrary",)),
    )(xyz1, xyz2t, x2, fc1_w, bias_col, bn_gamma[None, :], bn_beta[None, :])
    return out


# speedup vs baseline: 1.1000x; 1.1000x over previous
import jax
import jax.numpy as jnp
import numpy as np
from jax.experimental import pallas as pl
from jax.experimental.pallas import tpu as pltpu


def _fused_kernel(b_dim, nelem,
                  xyz1t_ref, xyz2t_ref, x2_ref, w_ref, b_ref, g_ref, bt_ref,
                  out_ref, yt_scr, sum_scr, sq_scr):
    i = pl.program_id(0)

    @pl.when(i < b_dim)
    def _compute():
        xyz1t = xyz1t_ref[0]
        xyz2t = xyz2t_ref[0]
        n1 = xyz1t.shape[1]
        n2 = xyz2t.shape[1]
        ab = jax.lax.dot_general(
            xyz1t.astype(jnp.bfloat16), xyz2t.astype(jnp.bfloat16),
            (((0,), (0,)), ((), ())), preferred_element_type=jnp.float32)
        b2 = (xyz2t[0] * xyz2t[0] + xyz2t[1] * xyz2t[1]
              + xyz2t[2] * xyz2t[2])
        a2 = (xyz1t[0] * xyz1t[0] + xyz1t[1] * xyz1t[1]
              + xyz1t[2] * xyz1t[2])[None, :].T
        d_sel = (-2.0 * ab + a2) + b2[None, :]

        cols = jax.lax.broadcasted_iota(
            jnp.int32, (n1, n2), 1).astype(jnp.float32)
        big = jnp.float32(n2)
        inf = jnp.float32(np.inf)
        dcur = d_sel
        recips = []
        for _ in range(3):
            dk = jnp.min(dcur, axis=1, keepdims=True)
            ik = jnp.min(jnp.where(dcur == dk, cols, big), axis=1,
                         keepdims=True)
            recips.append(1.0 / (dk + 1e-8))
            dcur = jnp.where(cols == ik, inf, dcur)
        inv_norm = 1.0 / (recips[0] + recips[1] + recips[2])
        r_wide = 1.0 / (d_sel + 1e-8)
        s_mat = jnp.where(dcur == inf, r_wide * inv_norm, 0.0)

        a_mat = jax.lax.dot_general(
            w_ref[...].astype(jnp.bfloat16), s_mat.astype(jnp.bfloat16),
            (((1,), (0,)), ((), ())),
            preferred_element_type=jnp.float32)
        yt = jax.lax.dot_general(
            a_mat.astype(jnp.bfloat16), x2_ref[0].astype(jnp.bfloat16),
            (((1,), (0,)), ((), ())),
            preferred_element_type=jnp.float32)
        yt = yt + b_ref[...]
        yt_scr[i] = yt
        sum_scr[pl.ds(i, 1), :] = jnp.sum(yt, axis=0, keepdims=True)
        sq_scr[pl.ds(i, 1), :] = jnp.sum(yt * yt, axis=0, keepdims=True)

    @pl.when(i >= b_dim)
    def _bn():
        bidx = i - b_dim
        s = jnp.sum(sum_scr[...], axis=0, keepdims=True)
        ss = jnp.sum(sq_scr[...], axis=0, keepdims=True)
        mean = s / nelem
        var = ss / nelem - mean * mean
        scale = g_ref[...] / jnp.sqrt(var + 1e-5)
        shift = bt_ref[...] - mean * scale
        out_ref[0] = jnp.maximum(yt_scr[bidx] * scale + shift, 0.0)


def kernel(x1, x2, xyz1, xyz2, fc1_w, fc1_b, bn_gamma, bn_beta):
    del x1
    b_dim, n2, c2 = x2.shape
    n1 = xyz1.shape[1]
    out_ch = fc1_w.shape[0]

    xyz1t = jnp.swapaxes(xyz1, 1, 2)
    xyz2t = jnp.swapaxes(xyz2, 1, 2)
    bias_col = fc1_b[:, None]
    nelem = float(b_dim * out_ch)

    def body(*refs):
        return _fused_kernel(b_dim, nelem, *refs)

    last = b_dim - 1
    out = pl.pallas_call(
        body,
        grid=(2 * b_dim,),
        in_specs=[
            pl.BlockSpec((1, 3, n1), lambda i: (jnp.minimum(i, last), 0, 0)),
            pl.BlockSpec((1, 3, n2), lambda i: (jnp.minimum(i, last), 0, 0)),
            pl.BlockSpec((1, n2, c2), lambda i: (jnp.minimum(i, last), 0, 0)),
            pl.BlockSpec((out_ch, n1), lambda i: (0, 0)),
            pl.BlockSpec((out_ch, 1), lambda i: (0, 0)),
            pl.BlockSpec((1, c2), lambda i: (0, 0)),
            pl.BlockSpec((1, c2), lambda i: (0, 0)),
        ],
        out_specs=pl.BlockSpec(
            (1, out_ch, c2),
            lambda i: (jnp.maximum(i - (last + 1), 0), 0, 0)),
        out_shape=jax.ShapeDtypeStruct((b_dim, out_ch, c2), jnp.float32),
        scratch_shapes=[
            pltpu.VMEM((b_dim, out_ch, c2), jnp.float32),
            pltpu.VMEM((b_dim, c2), jnp.float32),
            pltpu.VMEM((b_dim, c2), jnp.float32),
        ],
        compiler_params=pltpu.CompilerParams(
            dimension_semantics=("arbitrary",)),
    )(xyz1t, xyz2t, x2, fc1_w, bias_col, bn_gamma[None, :], bn_beta[None, :])
    return out
